# baseline (speedup 1.0000x reference)
import jax
import jax.numpy as jnp
from jax.experimental import pallas as pl
from jax.experimental.pallas import tpu as pltpu

N = 10000
E = 2000
D = 256
ALPHA = 0.2
BN = 1000
NB = N // BN
LOG2E = 1.4426950408889634


def _fused_kernel(x_ref, h_ref, w1_ref, w1a_ref, w1e_ref, w2_ref, w2e_ref,
                  w2a_ref, a1hi_ref, a1blo_ref, a1bhi_ref, a2blo_ref,
                  a2bhi_ref, c1_ref, a1lo_ref,
                  node_ref, edge2_ref,
                  hb_s, sacc_s, sz_s, p1_s, e1b_s, e2b_s, q1_s, q2_s, v2b_s):
    p = pl.program_id(0)
    i = pl.program_id(1)
    f32 = jnp.float32
    bf16 = jnp.bfloat16
    dn = (((0,), (0,)), ((), ()))

    @pl.when(p == 0)
    def _phase0():
        @pl.when(i == 0)
        def _():
            sacc_s[...] = jnp.zeros_like(sacc_s)
            sz_s[...] = jnp.zeros_like(sz_s)

        x = x_ref[...]
        h = h_ref[...]
        w1a = w1a_ref[...]
        v1a = jnp.dot(w1a, a1hi_ref[...], preferred_element_type=f32)
        v1b = jnp.dot(w1a, a1blo_ref[...], preferred_element_type=f32)
        c0 = jnp.sum(c1_ref[...] * a1lo_ref[...])
        s1v = jnp.dot(x, v1a, preferred_element_type=f32) + c0
        u = jnp.exp(jnp.maximum(s1v, ALPHA * s1v))
        p1 = jnp.dot(x, v1b, preferred_element_type=f32) * LOG2E
        p1_s[i] = jax.lax.transpose(p1, (1, 0))
        hb = h.astype(bf16)
        hb_s[i] = h.astype(jnp.int8)
        xt = jnp.dot(x, w1_ref[...], preferred_element_type=f32)
        t = u * xt
        th = t.astype(bf16)
        tl = (t - th.astype(f32)).astype(bf16)
        sacc_s[...] += (
            jax.lax.dot_general(th, hb, dn, preferred_element_type=f32)
            + jax.lax.dot_general(tl, hb, dn, preferred_element_type=f32))
        sz_s[...] += jax.lax.dot_general(u, h, dn, preferred_element_type=f32)

    @pl.when(p == 1)
    def _phase1():
        @pl.when(i == 0)
        def _():
            edge1t = sacc_s[...] * (1.0 / sz_s[...])
            e1b_s[...] = jax.lax.transpose(edge1t.astype(bf16), (1, 0))
            w1v = jnp.dot(w1e_ref[...], a1bhi_ref[...],
                          preferred_element_type=f32)
            q1_s[...] = jax.lax.dot_general(
                w1v, edge1t, dn, preferred_element_type=f32) * LOG2E
            edge2t = jax.lax.dot_general(w2_ref[...], edge1t, dn,
                                         preferred_element_type=f32)
            edge2_ref[...] = jax.lax.transpose(edge2t, (1, 0))
            e2b_s[...] = jax.lax.transpose(edge2t.astype(bf16), (1, 0))
            w2v = jnp.dot(w2e_ref[...], a2bhi_ref[...],
                          preferred_element_type=f32)
            q2_s[...] = jax.lax.dot_general(
                w2v, edge2t, dn, preferred_element_type=f32) * LOG2E
            v2b_s[...] = jnp.dot(w2a_ref[...], a2blo_ref[...],
                                 preferred_element_type=f32) * LOG2E

        ones = jnp.ones((E, 1), bf16)
        h = hb_s[i].astype(bf16)
        p1 = jax.lax.transpose(p1_s[i], (1, 0))
        w = p1 + q1_s[...]
        e = (jnp.exp2(jnp.maximum(w, ALPHA * w)) * h).astype(bf16)
        z1 = jnp.dot(e, ones, preferred_element_type=f32)
        node1 = jnp.dot(e, e1b_s[...],
                        preferred_element_type=f32) * (1.0 / z1)
        p2 = jnp.dot(node1, v2b_s[...], preferred_element_type=f32)
        w2s = p2 + q2_s[...]
        e2 = (jnp.exp2(jnp.maximum(w2s, ALPHA * w2s)) * h).astype(bf16)
        z2 = jnp.dot(e2, ones, preferred_element_type=f32)
        node_ref[...] = jnp.dot(e2, e2b_s[...],
                                preferred_element_type=f32) * (1.0 / z2)


def kernel(x, H, W1, W1a, W1e, a1, a1b, c1, W2, W2a, W2e, a2, a2b, c2):
    f32 = jnp.float32
    bf16 = jnp.bfloat16
    a1hi = a1[D:].reshape(D, 1)
    a1lo = a1[:D].reshape(1, D)
    a1blo = a1b[:D].reshape(D, 1)
    a1bhi = a1b[D:].reshape(D, 1)
    a2blo = a2b[:D].reshape(D, 1)
    a2bhi = a2b[D:].reshape(D, 1)
    c1r = c1.reshape(1, D)

    stream = lambda p, i: (jnp.where(p == 0, i, 0), 0)
    const = lambda p, i: (0, 0)
    outmap = lambda p, i: (jnp.where(p == 1, i, 0), 0)

    node2, edge2 = pl.pallas_call(
        _fused_kernel,
        grid=(2, NB),
        in_specs=[pl.BlockSpec((BN, D), stream),
                  pl.BlockSpec((BN, E), stream),
                  pl.BlockSpec((D, D), const),
                  pl.BlockSpec((D, D), const),
                  pl.BlockSpec((D, D), const),
                  pl.BlockSpec((D, D), const),
                  pl.BlockSpec((D, D), const),
                  pl.BlockSpec((D, D), const),
                  pl.BlockSpec((D, 1), const),
                  pl.BlockSpec((D, 1), const),
                  pl.BlockSpec((D, 1), const),
                  pl.BlockSpec((D, 1), const),
                  pl.BlockSpec((D, 1), const),
                  pl.BlockSpec((1, D), const),
                  pl.BlockSpec((1, D), const)],
        out_specs=(pl.BlockSpec((BN, D), outmap),
                   pl.BlockSpec((E, D), const)),
        out_shape=(jax.ShapeDtypeStruct((N, D), f32),
                   jax.ShapeDtypeStruct((E, D), f32)),
        scratch_shapes=[pltpu.VMEM((NB, BN, E), jnp.int8),
                        pltpu.VMEM((D, E), f32),
                        pltpu.VMEM((1, E), f32),
                        pltpu.VMEM((NB, 1, BN), f32),
                        pltpu.VMEM((E, D), bf16),
                        pltpu.VMEM((E, D), bf16),
                        pltpu.VMEM((1, E), f32),
                        pltpu.VMEM((1, E), f32),
                        pltpu.VMEM((D, 1), f32)],
    )(x, H, W1, W1a, W1e, W2, W2e, W2a, a1hi, a1blo, a1bhi, a2blo, a2bhi,
      c1r, a1lo)

    return (node2, edge2)


# speedup vs baseline: 1.0013x; 1.0013x over previous
import jax
import jax.numpy as jnp
from jax.experimental import pallas as pl
from jax.experimental.pallas import tpu as pltpu

N = 10000
E = 2000
D = 256
ALPHA = 0.2
BN = 1000
NB = N // BN
LOG2E = 1.4426950408889634


def _fused_kernel(x_ref, h_ref, w1_ref, w1a_ref, w1e_ref, w2_ref, w2e_ref,
                  w2a_ref, a1hi_ref, a1blo_ref, a1bhi_ref, a2blo_ref,
                  a2bhi_ref, c1_ref, a1lo_ref,
                  node_ref, edge2_ref,
                  hb_s, sacc_s, sz_s, p1_s, e1b_s, e2b_s, q1_s, q2_s, v2b_s):
    p = pl.program_id(0)
    i = pl.program_id(1)
    f32 = jnp.float32
    bf16 = jnp.bfloat16
    dn = (((0,), (0,)), ((), ()))

    @pl.when(p == 0)
    def _phase0():
        @pl.when(i == 0)
        def _():
            sacc_s[...] = jnp.zeros_like(sacc_s)
            sz_s[...] = jnp.zeros_like(sz_s)

        x = x_ref[...]
        h = h_ref[...]
        w1a = w1a_ref[...]
        v1a = jnp.dot(w1a, a1hi_ref[...], preferred_element_type=f32)
        v1b = jnp.dot(w1a, a1blo_ref[...], preferred_element_type=f32)
        c0 = jnp.sum(c1_ref[...] * a1lo_ref[...])
        s1v = jnp.dot(x, v1a, preferred_element_type=f32) + c0
        u = jnp.exp(jnp.maximum(s1v, ALPHA * s1v))
        p1 = jnp.dot(x, v1b, preferred_element_type=f32) * LOG2E
        p1_s[i] = jax.lax.transpose(p1, (1, 0))
        hb = h.astype(bf16)
        hb_s[i] = h.astype(jnp.int8)
        xt = jnp.dot(x, w1_ref[...], preferred_element_type=f32)
        t = u * xt
        th = t.astype(bf16)
        tl = (t - th.astype(f32)).astype(bf16)
        sacc_s[...] += (
            jax.lax.dot_general(th, hb, dn, preferred_element_type=f32)
            + jax.lax.dot_general(tl, hb, dn, preferred_element_type=f32))
        sz_s[...] += jax.lax.dot_general(u, h, dn, preferred_element_type=f32)

    @pl.when(p == 1)
    def _phase1():
        @pl.when(i == 0)
        def _():
            edge1t = sacc_s[...] * (1.0 / sz_s[...])
            e1b_s[...] = jax.lax.transpose(edge1t.astype(bf16), (1, 0))
            w1v = jnp.dot(w1e_ref[...], a1bhi_ref[...],
                          preferred_element_type=f32)
            q1_s[...] = jax.lax.dot_general(
                w1v, edge1t, dn, preferred_element_type=f32) * LOG2E
            edge2t = jax.lax.dot_general(w2_ref[...], edge1t, dn,
                                         preferred_element_type=f32)
            edge2_ref[...] = jax.lax.transpose(edge2t, (1, 0))
            e2b_s[...] = jax.lax.transpose(edge2t.astype(bf16), (1, 0))
            w2v = jnp.dot(w2e_ref[...], a2bhi_ref[...],
                          preferred_element_type=f32)
            q2_s[...] = jax.lax.dot_general(
                w2v, edge2t, dn, preferred_element_type=f32) * LOG2E
            v2b_s[...] = jnp.dot(w2a_ref[...], a2blo_ref[...],
                                 preferred_element_type=f32) * LOG2E

        ones = jnp.ones((E, 1), bf16)
        hf = hb_s[i].astype(f32)
        p1 = jax.lax.transpose(p1_s[i], (1, 0))
        w = p1 + q1_s[...]
        e = (jnp.exp2(jnp.maximum(w, ALPHA * w)) * hf).astype(bf16)
        z1 = jnp.dot(e, ones, preferred_element_type=f32)
        node1 = jnp.dot(e, e1b_s[...],
                        preferred_element_type=f32) * (1.0 / z1)
        p2 = jnp.dot(node1, v2b_s[...], preferred_element_type=f32)
        w2s = p2 + q2_s[...]
        e2 = (jnp.exp2(jnp.maximum(w2s, ALPHA * w2s)) * hf).astype(bf16)
        z2 = jnp.dot(e2, ones, preferred_element_type=f32)
        node_ref[...] = jnp.dot(e2, e2b_s[...],
                                preferred_element_type=f32) * (1.0 / z2)


def kernel(x, H, W1, W1a, W1e, a1, a1b, c1, W2, W2a, W2e, a2, a2b, c2):
    f32 = jnp.float32
    bf16 = jnp.bfloat16
    a1hi = a1[D:].reshape(D, 1)
    a1lo = a1[:D].reshape(1, D)
    a1blo = a1b[:D].reshape(D, 1)
    a1bhi = a1b[D:].reshape(D, 1)
    a2blo = a2b[:D].reshape(D, 1)
    a2bhi = a2b[D:].reshape(D, 1)
    c1r = c1.reshape(1, D)

    stream = lambda p, i: (jnp.where(p == 0, i, 0), 0)
    const = lambda p, i: (0, 0)
    outmap = lambda p, i: (jnp.where(p == 1, i, 0), 0)

    node2, edge2 = pl.pallas_call(
        _fused_kernel,
        grid=(2, NB),
        in_specs=[pl.BlockSpec((BN, D), stream),
                  pl.BlockSpec((BN, E), stream),
                  pl.BlockSpec((D, D), const),
                  pl.BlockSpec((D, D), const),
                  pl.BlockSpec((D, D), const),
                  pl.BlockSpec((D, D), const),
                  pl.BlockSpec((D, D), const),
                  pl.BlockSpec((D, D), const),
                  pl.BlockSpec((D, 1), const),
                  pl.BlockSpec((D, 1), const),
                  pl.BlockSpec((D, 1), const),
                  pl.BlockSpec((D, 1), const),
                  pl.BlockSpec((D, 1), const),
                  pl.BlockSpec((1, D), const),
                  pl.BlockSpec((1, D), const)],
        out_specs=(pl.BlockSpec((BN, D), outmap),
                   pl.BlockSpec((E, D), const)),
        out_shape=(jax.ShapeDtypeStruct((N, D), f32),
                   jax.ShapeDtypeStruct((E, D), f32)),
        scratch_shapes=[pltpu.VMEM((NB, BN, E), jnp.int8),
                        pltpu.VMEM((D, E), f32),
                        pltpu.VMEM((1, E), f32),
                        pltpu.VMEM((NB, 1, BN), f32),
                        pltpu.VMEM((E, D), bf16),
                        pltpu.VMEM((E, D), bf16),
                        pltpu.VMEM((1, E), f32),
                        pltpu.VMEM((1, E), f32),
                        pltpu.VMEM((D, 1), f32)],
    )(x, H, W1, W1a, W1e, W2, W2e, W2a, a1hi, a1blo, a1bhi, a2blo, a2bhi,
      c1r, a1lo)

    return (node2, edge2)
